# baseline (speedup 1.0000x reference)
import jax
import jax.numpy as jnp
from jax import lax
from jax.experimental import pallas as pl
from jax.experimental.pallas import tpu as pltpu
from jax.experimental.pallas import tpu_sc as plsc

SEQ_LEN = 512
CLS_ID = 101
SEP_ID = 102
PAD_ID = 0
BATCH = 16
TOTAL_TOKENS = 32768

_L = 16
_NC = 2
_NS = 16
_NW = _NC * _NS
_CHUNK = (BATCH * SEQ_LEN) // _NW
_WIN = 288
_WBUF = 832


def _body(cu_hbm, flat_hbm, out_hbm, cu_v, win_v, ov, sem):
    wid = lax.axis_index("s") * _NC + lax.axis_index("c")
    b = wid // 2
    h = wid % 2
    base = h * _CHUNK

    pltpu.sync_copy(cu_hbm.at[pl.ds(0, _L)], cu_v)
    iv = lax.iota(jnp.int32, _L)
    v0 = cu_v[...]
    pick_idx = jnp.minimum(
        jnp.full((_L,), b, jnp.int32) + jnp.where(iv == 0, 0, 1), _L - 1
    )
    picked = lax.gather(
        v0,
        pick_idx[:, None],
        lax.GatherDimensionNumbers(
            offset_dims=(), collapsed_slice_dims=(0,), start_index_map=(0,)
        ),
        slice_sizes=(1,),
        mode=lax.GatherScatterMode.PROMISE_IN_BOUNDS,
    )
    s = picked[0]
    e = jnp.where(b == BATCH - 1, jnp.int32(TOTAL_TOKENS), picked[1])
    t1 = jnp.minimum(e - s, SEQ_LEN - 2) + 1

    aoff = jnp.clip(s + base - 8, 0, TOTAL_TOKENS - _WIN) & ~jnp.int32(7)
    aoff = pl.multiple_of(aoff, 8)
    d8 = s + base + 7 - aoff
    pltpu.async_copy(
        flat_hbm.at[pl.ds(aoff, _WIN)], win_v.at[pl.ds(8, _WIN)], sem
    ).wait()

    for j in range(_CHUNK // _L):
        pos = base + j * _L + iv
        tok = win_v[pl.ds(d8 + j * _L, _L)]
        m = pos <= t1
        out = jnp.where(m, jnp.where(pos == t1, jnp.int32(SEP_ID), tok), jnp.int32(PAD_ID))
        if j == 0:
            out = jnp.where(pos == 0, jnp.int32(CLS_ID), out)
        ov[pl.ds(j * _L, _L)] = out
        ov[pl.ds(_CHUNK + j * _L, _L)] = jnp.where(m, jnp.int32(1), jnp.int32(0))

    pltpu.async_copy(ov, out_hbm.at[wid], sem).wait()


def kernel(flat_tokens, cu_seqlens):
    mesh = plsc.VectorSubcoreMesh(core_axis_name="c", subcore_axis_name="s")
    packed = pl.kernel(
        _body,
        mesh=mesh,
        out_type=jax.ShapeDtypeStruct((_NW, 2 * _CHUNK), jnp.int32),
        scratch_types=[
            pltpu.VMEM((_L,), jnp.int32),
            pltpu.VMEM((_WBUF,), jnp.int32),
            pltpu.VMEM((2 * _CHUNK,), jnp.int32),
            pltpu.SemaphoreType.DMA,
        ],
    )(cu_seqlens.astype(jnp.int32), flat_tokens.astype(jnp.int32))
    token_ids = packed[:, :_CHUNK].reshape(BATCH, SEQ_LEN)
    padding_mask = packed[:, _CHUNK:].reshape(BATCH, SEQ_LEN).astype(jnp.bool_)
    return token_ids, padding_mask


# speedup vs baseline: 1.0070x; 1.0070x over previous
import jax
import jax.numpy as jnp
from jax import lax
from jax.experimental import pallas as pl
from jax.experimental.pallas import tpu as pltpu
from jax.experimental.pallas import tpu_sc as plsc

SEQ_LEN = 512
CLS_ID = 101
SEP_ID = 102
PAD_ID = 0
BATCH = 16
TOTAL_TOKENS = 32768

_L = 16
_NC = 2
_NS = 16
_NW = _NC * _NS
_CHUNK = (BATCH * SEQ_LEN) // _NW
_SLICE = TOTAL_TOKENS // _NS
_WIN = 288
_WBUF = 832


def _body(cu_hbm, flat_hbm, out_hbm, flat_sh, cu_v, win_v, ov, sem, sem2):
    sid = lax.axis_index("s")
    wid = sid * _NC + lax.axis_index("c")
    b = wid // 2
    h = wid % 2
    base = h * _CHUNK

    c_cu = pltpu.async_copy(cu_hbm.at[pl.ds(0, _L)], cu_v, sem)
    c_fl = pltpu.async_copy(
        flat_hbm.at[pl.ds(sid * _SLICE, _SLICE)],
        flat_sh.at[pl.ds(sid * _SLICE, _SLICE)],
        sem2,
    )
    c_cu.wait()

    iv = lax.iota(jnp.int32, _L)
    v0 = cu_v[...]
    pick_idx = jnp.minimum(
        jnp.full((_L,), b, jnp.int32) + jnp.where(iv == 0, 0, 1), _L - 1
    )
    picked = lax.gather(
        v0,
        pick_idx[:, None],
        lax.GatherDimensionNumbers(
            offset_dims=(), collapsed_slice_dims=(0,), start_index_map=(0,)
        ),
        slice_sizes=(1,),
        mode=lax.GatherScatterMode.PROMISE_IN_BOUNDS,
    )
    s = picked[0]
    e = jnp.where(b == BATCH - 1, jnp.int32(TOTAL_TOKENS), picked[1])
    t1 = jnp.minimum(e - s, SEQ_LEN - 2) + 1

    aoff = jnp.clip(s + base - 8, 0, TOTAL_TOKENS - _WIN) & ~jnp.int32(7)
    aoff = pl.multiple_of(aoff, 8)
    d8 = s + base + 7 - aoff

    c_fl.wait()
    plsc.subcore_barrier()

    pltpu.async_copy(
        flat_sh.at[pl.ds(aoff, _WIN)], win_v.at[pl.ds(8, _WIN)], sem
    ).wait()

    for j in range(_CHUNK // _L):
        pos = base + j * _L + iv
        tok = win_v[pl.ds(d8 + j * _L, _L)]
        m = pos <= t1
        out = jnp.where(m, jnp.where(pos == t1, jnp.int32(SEP_ID), tok), jnp.int32(PAD_ID))
        if j == 0:
            out = jnp.where(pos == 0, jnp.int32(CLS_ID), out)
        ov[pl.ds(j * _L, _L)] = out
        ov[pl.ds(_CHUNK + j * _L, _L)] = jnp.where(m, jnp.int32(1), jnp.int32(0))

    pltpu.async_copy(ov, out_hbm.at[wid], sem).wait()


def kernel(flat_tokens, cu_seqlens):
    mesh = plsc.VectorSubcoreMesh(core_axis_name="c", subcore_axis_name="s")
    packed = pl.kernel(
        _body,
        mesh=mesh,
        out_type=jax.ShapeDtypeStruct((_NW, 2 * _CHUNK), jnp.int32),
        scratch_types=[
            pltpu.MemorySpace.VMEM_SHARED((TOTAL_TOKENS,), jnp.int32),
            pltpu.VMEM((_L,), jnp.int32),
            pltpu.VMEM((_WBUF,), jnp.int32),
            pltpu.VMEM((2 * _CHUNK,), jnp.int32),
            pltpu.SemaphoreType.DMA,
            pltpu.SemaphoreType.DMA,
        ],
    )(cu_seqlens.astype(jnp.int32), flat_tokens.astype(jnp.int32))
    token_ids = packed[:, :_CHUNK].reshape(BATCH, SEQ_LEN)
    padding_mask = packed[:, _CHUNK:].reshape(BATCH, SEQ_LEN).astype(jnp.bool_)
    return token_ids, padding_mask
